# Initial kernel scaffold; baseline (speedup 1.0000x reference)
import jax
import jax.numpy as jnp
from jax.experimental import pallas as pl


def kernel(features, nodes, labels, edge_index1, edge_index2, edge_index3, W_mlp, b_mlp, W1, W2, W3, Ws1, Ws2, Ws3, weight):
    raise NotImplementedError("write your pallas kernel here")


# speedup vs baseline: 5.1880x; 5.1880x over previous
import functools

import jax
import jax.numpy as jnp
from jax import lax
from jax.experimental import pallas as pl
from jax.experimental.pallas import tpu as pltpu
from jax.experimental.pallas import tpu_sc as plsc

N = 50000
FEAT = 128
MLPD = 64
B = 4096
E = 800000

D = 128
NC = 2
NS = 16
NW = NC * NS
C = 1280
NCHUNK = E // C
K = 128
BP = B + 128
ROWS_PER_TILE = BP // NS
RELS = 3


def _mlp_body(x_ref, w_ref, b_ref, out_ref):
    y = jnp.dot(x_ref[...], w_ref[...], preferred_element_type=jnp.float32,
                precision=lax.Precision.HIGHEST)
    y = jnp.maximum(y + b_ref[...], 0.0)
    rows = y.shape[0]
    ones = jnp.ones((rows, 1), jnp.float32)
    pad = jnp.zeros((rows, D - MLPD - 1), jnp.float32)
    out_ref[...] = jnp.concatenate([y, ones, pad], axis=1)


def _mlp(features, W_mlp, b2d):
    blk = 2000
    grid = N // blk
    return pl.pallas_call(
        _mlp_body,
        grid=(grid,),
        in_specs=[
            pl.BlockSpec((blk, FEAT), lambda i: (i, 0)),
            pl.BlockSpec((FEAT, MLPD), lambda i: (0, 0)),
            pl.BlockSpec((1, MLPD), lambda i: (0, 0)),
        ],
        out_specs=pl.BlockSpec((blk, D), lambda i: (i, 0)),
        out_shape=jax.ShapeDtypeStruct((N, D), jnp.float32),
    )(features, W_mlp, b2d)


ZROWS = ROWS_PER_TILE // 3


def _sc_agg_body(f1aug, inv_hbm, e1, e2, e3, out,
                 inv_v, dst_v, src_v, pend_p, pend_s, pidx, sidx, rowbuf,
                 zbuf, sem, acc):
    c = lax.axis_index("c")
    s = lax.axis_index("s")
    wid = c * NS + s
    edges = [e1, e2, e3]
    base_z = s * ROWS_PER_TILE

    def _zrow(i, _):
        def _zcol(j, __):
            zbuf[i, pl.ds(j * 16, 16)] = jnp.zeros((16,), jnp.float32)
            return 0
        return lax.fori_loop(0, D // 16, _zcol, 0)
    lax.fori_loop(0, ZROWS, _zrow, 0)

    pltpu.sync_copy(inv_hbm, inv_v)

    trash16 = jnp.full((16,), B, jnp.int32)
    zero16 = jnp.zeros((16,), jnp.int32)

    for r in range(RELS):
        e = edges[r]

        for z in range(3):
            pltpu.sync_copy(zbuf, acc.at[pl.ds(base_z + z * ZROWS, ZROWS)])
        plsc.subcore_barrier()

        def chunk_body(it, _, e=e, acc=acc):
            t = wid + it * NW
            base = t * C
            pltpu.sync_copy(e.at[1, pl.ds(base, C)], dst_v)
            pltpu.sync_copy(e.at[0, pl.ds(base, C)], src_v)

            def vbody(j, cnt):
                dvec = dst_v[pl.ds(j * 16, 16)]
                p = plsc.load_gather(inv_v, [dvec])
                m = p >= 0
                svec = src_v[pl.ds(j * 16, 16)]
                plsc.store_compressed(pend_p.at[pl.ds(cnt, 16)], p, mask=m)
                plsc.store_compressed(pend_s.at[pl.ds(cnt, 16)], svec, mask=m)
                return cnt + jnp.sum(m.astype(jnp.int32))
            cnt = lax.fori_loop(0, C // 16, vbody, 0)

            for j in range(K // 16):
                pend_p[pl.ds(cnt + j * 16, 16)] = trash16
                pend_s[pl.ds(cnt + j * 16, 16)] = zero16

            nflush = (cnt + K - 1) // K

            def fbody(f, __):
                off = f * K
                def cpy(j, ___):
                    pidx[pl.ds(j * 16, 16)] = pend_p[pl.ds(off + j * 16, 16)]
                    sidx[pl.ds(j * 16, 16)] = pend_s[pl.ds(off + j * 16, 16)]
                    return 0
                lax.fori_loop(0, K // 16, cpy, 0)
                pltpu.async_copy(f1aug.at[sidx], rowbuf, sem).wait()
                pltpu.sync_copy(rowbuf, acc.at[pidx], add=True)
                return 0
            lax.fori_loop(0, nflush, fbody, 0)
            return 0

        nmy = (NCHUNK - 1 - wid) // NW + 1
        lax.fori_loop(0, nmy, chunk_body, 0)

        plsc.subcore_barrier()
        pltpu.sync_copy(
            acc.at[pl.ds(base_z, ROWS_PER_TILE)],
            out.at[c * RELS + r, pl.ds(base_z, ROWS_PER_TILE)])
        plsc.subcore_barrier()


def _sc_agg(f1aug, inv, e1, e2, e3):
    mesh = plsc.VectorSubcoreMesh(core_axis_name="c", subcore_axis_name="s")
    fn = functools.partial(
        pl.kernel,
        out_type=jax.ShapeDtypeStruct((NC * RELS, BP, D), jnp.float32),
        mesh=mesh,
        compiler_params=pltpu.CompilerParams(needs_layout_passes=False),
        scratch_types=[
            pltpu.VMEM((N,), jnp.int32),
            pltpu.VMEM((C,), jnp.int32),
            pltpu.VMEM((C,), jnp.int32),
            pltpu.VMEM((C + K + 16,), jnp.int32),
            pltpu.VMEM((C + K + 16,), jnp.int32),
            pltpu.VMEM((K,), jnp.int32),
            pltpu.VMEM((K,), jnp.int32),
            pltpu.VMEM((K, D), jnp.float32),
            pltpu.VMEM((ZROWS, D), jnp.float32),
            pltpu.SemaphoreType.DMA,
            pltpu.VMEM_SHARED((BP, D), jnp.float32),
        ],
    )(_sc_agg_body)
    return fn(f1aug, inv, e1, e2, e3)


def _sc_batch_body(parts, pb, nodes, f1aug, br_out, f1b_out,
                   idxv, nidx, buf, sem):
    c = lax.axis_index("c")
    s = lax.axis_index("s")
    wid = c * NS + s
    nb = B // NW
    base = wid * nb

    pltpu.sync_copy(pb.at[pl.ds(base, nb)], idxv)
    pltpu.sync_copy(nodes.at[pl.ds(base, nb)], nidx)

    pltpu.async_copy(f1aug.at[nidx], buf, sem).wait()
    pltpu.sync_copy(buf, f1b_out.at[pl.ds(base, nb)])

    for r in range(RELS):
        pltpu.async_copy(parts.at[r].at[idxv], buf, sem).wait()
        pltpu.async_copy(parts.at[RELS + r].at[idxv], buf, sem, add=True).wait()
        pltpu.sync_copy(buf, br_out.at[r, pl.ds(base, nb)])


def _sc_batch(parts, pb, nodes, f1aug):
    mesh = plsc.VectorSubcoreMesh(core_axis_name="c", subcore_axis_name="s")
    fn = functools.partial(
        pl.kernel,
        out_type=(jax.ShapeDtypeStruct((RELS, B, D), jnp.float32),
                  jax.ShapeDtypeStruct((B, D), jnp.float32)),
        mesh=mesh,
        compiler_params=pltpu.CompilerParams(needs_layout_passes=False),
        scratch_types=[
            pltpu.VMEM((B // NW,), jnp.int32),
            pltpu.VMEM((B // NW,), jnp.int32),
            pltpu.VMEM((B // NW, D), jnp.float32),
            pltpu.SemaphoreType.DMA,
        ],
    )(_sc_batch_body)
    return fn(parts, pb, nodes, f1aug)


def _head_body(f1b_ref, br_ref, lab_ref, w1, w2, w3, ws1, ws2, ws3, wt,
               comb_ref, loss_ref):
    i = pl.program_id(0)
    f1 = f1b_ref[:, :MLPD]
    lab = lab_ref[...]
    hs = [f1]
    loss = jnp.zeros((1, 1), jnp.float32)
    for r, (w, ws) in enumerate(((w1, ws1), (w2, ws2), (w3, ws3))):
        row = br_ref[r]
        ssum = row[:, :MLPD]
        deg = row[:, MLPD:MLPD + 1]
        neigh = ssum / jnp.maximum(deg, 1.0)
        cat = jnp.concatenate([f1, neigh], axis=1)
        h = jnp.maximum(
            jnp.dot(cat, w[...], preferred_element_type=jnp.float32,
                    precision=lax.Precision.HIGHEST), 0.0)
        hs.append(h)
        logits = jnp.dot(h, ws[...], preferred_element_type=jnp.float32,
                         precision=lax.Precision.HIGHEST)
        l0 = logits[:, 0:1]
        l1 = logits[:, 1:2]
        m = jnp.maximum(l0, l1)
        lse = m + jnp.log(jnp.exp(l0 - m) + jnp.exp(l1 - m))
        ll = jnp.where(lab == 0, l0, l1)
        loss = loss + jnp.sum(lse - ll, keepdims=True).reshape(1, 1) / B
    cat2 = jnp.concatenate(hs, axis=1)
    comb_ref[...] = jnp.maximum(
        jnp.dot(cat2, wt[...], preferred_element_type=jnp.float32,
                precision=lax.Precision.HIGHEST), 0.0)

    @pl.when(i == 0)
    def _():
        loss_ref[...] = jnp.zeros((1, 1), jnp.float32)
    loss_ref[...] += loss


def _tc_head(f1b, br, lab2d, W1, W2, W3, Ws1, Ws2, Ws3, weight):
    blk = 1024
    grid = B // blk
    return pl.pallas_call(
        _head_body,
        grid=(grid,),
        in_specs=[
            pl.BlockSpec((blk, D), lambda i: (i, 0)),
            pl.BlockSpec((RELS, blk, D), lambda i: (0, i, 0)),
            pl.BlockSpec((blk, 1), lambda i: (i, 0)),
            pl.BlockSpec((2 * MLPD, MLPD), lambda i: (0, 0)),
            pl.BlockSpec((2 * MLPD, MLPD), lambda i: (0, 0)),
            pl.BlockSpec((2 * MLPD, MLPD), lambda i: (0, 0)),
            pl.BlockSpec((MLPD, 2), lambda i: (0, 0)),
            pl.BlockSpec((MLPD, 2), lambda i: (0, 0)),
            pl.BlockSpec((MLPD, 2), lambda i: (0, 0)),
            pl.BlockSpec((MLPD + 3 * MLPD, MLPD), lambda i: (0, 0)),
        ],
        out_specs=(pl.BlockSpec((blk, MLPD), lambda i: (i, 0)),
                   pl.BlockSpec((1, 1), lambda i: (0, 0))),
        out_shape=(jax.ShapeDtypeStruct((B, MLPD), jnp.float32),
                   jax.ShapeDtypeStruct((1, 1), jnp.float32)),
    )(f1b, br, lab2d, W1, W2, W3, Ws1, Ws2, Ws3, weight)


def kernel(features, nodes, labels, edge_index1, edge_index2, edge_index3,
           W_mlp, b_mlp, W1, W2, W3, Ws1, Ws2, Ws3, weight):
    nodes = nodes.astype(jnp.int32)
    f1aug = _mlp(features, W_mlp, b_mlp.reshape(1, MLPD))
    inv = jnp.full((N,), -1, jnp.int32).at[nodes].set(
        jnp.arange(B, dtype=jnp.int32))
    pb = inv[nodes]
    parts = _sc_agg(f1aug, inv,
                    edge_index1.astype(jnp.int32),
                    edge_index2.astype(jnp.int32),
                    edge_index3.astype(jnp.int32))
    br, f1b = _sc_batch(parts, pb, nodes, f1aug)
    comb, loss = _tc_head(f1b, br, labels.reshape(B, 1).astype(jnp.int32),
                          W1, W2, W3, Ws1, Ws2, Ws3, weight)
    return comb.T, f1b[:, :MLPD].T, loss.reshape(())
